# Initial kernel scaffold; baseline (speedup 1.0000x reference)
import jax
import jax.numpy as jnp
from jax.experimental import pallas as pl


def kernel(input_ids, weight):
    raise NotImplementedError("write your pallas kernel here")


# speedup vs baseline: 4.6769x; 4.6769x over previous
import functools

import jax
import jax.numpy as jnp
from jax import lax
from jax.experimental import pallas as pl
from jax.experimental.pallas import tpu as pltpu
from jax.experimental.pallas import tpu_sc as plsc

_DIM = 64
_NC = 2
_NS = 16
_NW = _NC * _NS


@functools.lru_cache(maxsize=None)
def _emb_lookup(n_total, n_per_w, n_chunk, chunk):
  mesh = plsc.VectorSubcoreMesh(core_axis_name="c", subcore_axis_name="s")

  @functools.partial(
      pl.kernel,
      mesh=mesh,
      out_type=jax.ShapeDtypeStruct((n_total, _DIM), jnp.float32),
      compiler_params=pltpu.CompilerParams(use_tc_tiling_on_sc=False),
      scratch_types=[
          pltpu.VMEM((n_per_w,), jnp.int32),
          pltpu.VMEM((chunk, _DIM), jnp.float32),
          pltpu.SemaphoreType.DMA,
      ],
  )
  def k(idx_hbm, table_hbm, out_hbm, idx_v, buf, sem):
    wid = lax.axis_index("s") * _NC + lax.axis_index("c")
    base = wid * n_per_w
    pltpu.sync_copy(idx_hbm.at[pl.ds(base, n_per_w)], idx_v)
    for c in range(n_chunk):
      pltpu.async_copy(
          table_hbm.at[idx_v.at[pl.ds(c * chunk, chunk)]], buf, sem
      ).wait()
      pltpu.sync_copy(buf, out_hbm.at[pl.ds(base + c * chunk, chunk)])

  return k


@jax.jit
def kernel(input_ids, weight):
  b, h = input_ids.shape
  n = b * h
  idx = input_ids.reshape(n).astype(jnp.int32)
  n_per_w = n // _NW
  chunk = 1600
  n_chunk = n_per_w // chunk
  out = _emb_lookup(n, n_per_w, n_chunk, chunk)(idx, weight)
  return out.reshape(b, h, _DIM)
